# baseline (speedup 1.0000x reference)
import functools
import jax
import jax.numpy as jnp
from jax import lax
from jax.experimental import pallas as pl
from jax.experimental.pallas import tpu as pltpu
from jax.experimental.pallas import tpu_sc as plsc

PAD_ID = 0
EOS_ID = 2
BATCH = 4096
SEQ = 20
DIM = 64

NUM_CORES = 2
NUM_SUBCORES = 16
NW = NUM_CORES * NUM_SUBCORES
TOTAL = BATCH * SEQ
ROWS_PER_W = TOTAL // NW
CHUNK = 640
NCHUNK = ROWS_PER_W // CHUNK
NBUF = 2


_mesh = plsc.VectorSubcoreMesh(
    core_axis_name="c", subcore_axis_name="s",
    num_cores=NUM_CORES, num_subcores=NUM_SUBCORES)


@functools.partial(
    pl.kernel,
    mesh=_mesh,
    out_type=jax.ShapeDtypeStruct((TOTAL, DIM), jnp.float32),
    scratch_types=[
        pltpu.VMEM((CHUNK,), jnp.int32),
        pltpu.VMEM((NBUF, CHUNK, DIM), jnp.float32),
        pltpu.SemaphoreType.DMA((NBUF,)),
        pltpu.SemaphoreType.DMA((NBUF,)),
    ],
    compiler_params=pltpu.CompilerParams(use_tc_tiling_on_sc=False),
)
def _sc_gather(ids_hbm, table_hbm, out_hbm, idx_s, rows_v, gsem, osem):
    wid = lax.axis_index("s") * NUM_CORES + lax.axis_index("c")
    base = wid * ROWS_PER_W

    def do_chunk(j):
        b = j % NBUF
        pltpu.sync_copy(ids_hbm.at[wid, j], idx_s)

        def grp_start(g, carry):
            vec = idx_s[pl.ds(g * 16, 16)]
            for l in range(16):
                pltpu.async_copy(
                    table_hbm.at[pl.ds(vec[l], 1)],
                    rows_v.at[b, pl.ds(g * 16 + l, 1)],
                    gsem.at[b])
            return carry

        lax.fori_loop(0, CHUNK // 16, grp_start, 0)

        def row_wait(r, carry):
            pltpu.make_async_copy(
                table_hbm.at[pl.ds(0, 1)],
                rows_v.at[b, pl.ds(0, 1)],
                gsem.at[b]).wait()
            return carry

        lax.fori_loop(0, CHUNK, row_wait, 0)
        pltpu.async_copy(
            rows_v.at[b],
            out_hbm.at[pl.ds(base + j * CHUNK, CHUNK)], osem.at[b])

    def out_done(j):
        pltpu.make_async_copy(
            rows_v.at[j % NBUF],
            out_hbm.at[pl.ds(base + j * CHUNK, CHUNK)],
            osem.at[j % NBUF]).wait()

    for j in range(NCHUNK):
        if j >= NBUF:
            out_done(j - NBUF)
        do_chunk(j)
    for j in range(max(NCHUNK - NBUF, 0), NCHUNK):
        out_done(j)


def _mask_body(ids_ref, mask_ref, eos_ref):
    ids = ids_ref[...]
    mask_ref[...] = ids == PAD_ID
    eos_ref[...] = (ids == EOS_ID).astype(jnp.float32)


_mask_call = pl.pallas_call(
    _mask_body,
    out_shape=(
        jax.ShapeDtypeStruct((TOTAL // 128, 128), jnp.bool_),
        jax.ShapeDtypeStruct((TOTAL // 128, 128), jnp.float32),
    ),
)


def kernel(lookup_ids, embedding_matrix):
    flat = lookup_ids.reshape(-1)
    ids_sc = flat.reshape(NW, NCHUNK, CHUNK)
    gathered = _sc_gather(ids_sc, embedding_matrix)
    matrices = gathered.reshape(BATCH, SEQ, DIM)
    mask2d, eos2d = _mask_call(flat.reshape(TOTAL // 128, 128))
    mask = mask2d.reshape(BATCH, SEQ)
    eos = eos2d.reshape(BATCH, SEQ)
    return (matrices, mask, eos)
